# Initial kernel scaffold; baseline (speedup 1.0000x reference)
import jax
import jax.numpy as jnp
from jax.experimental import pallas as pl


def kernel(anchor, feat, grid_scaling, grid_offsets, hyper_feat, noise_feat, noise_scaling, noise_offsets, mlp_w, mlp_b, idx):
    raise NotImplementedError("write your pallas kernel here")


# speedup vs baseline: 2.3271x; 2.3271x over previous
import functools

import jax
import jax.numpy as jnp
from jax import lax
from jax.experimental import pallas as pl
from jax.experimental.pallas import tpu as pltpu
from jax.experimental.pallas import tpu_sc as plsc

N = 500000
B = 250000
FEAT = 50
NOFF = 10
OFFW = 3 * NOFF
OUTW = FEAT + 6 + OFFW
IN_DIM = 9 + 2 * FEAT

NC = 2
NS = 16
NW = NC * NS
L = 16

CW = 64
NCHUNKS = (N + CW - 1) // CW
TAIL_C = NCHUNKS - 1
TAIL_W = N - TAIL_C * CW
CPW = (NCHUNKS + NW - 1) // NW
WSLOTS = CPW * CW

IW = 2000
NIW = B // IW

NOISE_W = 128


def _group_body(g, li, winner_v, anchor_v, feat_v, scal_v, offs_v, hyper_v,
                noise_v, out_v, w_s):
    rows = g * L + lax.iota(jnp.int32, L)
    wb = w_s[pl.ds(4 * IN_DIM, L)]
    acc = [jnp.zeros((L,), jnp.float32) + wb[j] for j in range(3)]

    def dot_tbl(tbl, ncols, wrow0):
        for k in range(ncols):
            col = plsc.load_gather(tbl, [rows, jnp.full((L,), k, jnp.int32)])
            wk = w_s[pl.ds(4 * (wrow0 + k), L)]
            for j in range(3):
                acc[j] = col * wk[j] + acc[j]

    dot_tbl(anchor_v, 3, 0)
    dot_tbl(feat_v, FEAT, 3)
    dot_tbl(scal_v, 6, 3 + FEAT)
    dot_tbl(hyper_v, FEAT, 9 + FEAT)

    wv = winner_v[pl.ds(li * CW + g * L, L)]
    m01 = jnp.where(wv >= 0, 1.0, 0.0).astype(jnp.float32)
    q = []
    for j, scale in ((0, 1.0), (1, 0.001), (2, 0.2)):
        t = 1.0 - 2.0 / (jnp.exp(acc[j] * 2.0) + 1.0)
        q.append(jnp.maximum(scale * (1.0 + t), 1e-9))

    def hyb(tbl, ncols, noise_c0, out_c0, qj):
        for c in range(ncols):
            tc = plsc.load_gather(tbl, [rows, jnp.full((L,), c, jnp.int32)])
            nc = plsc.load_gather(
                noise_v, [rows, jnp.full((L,), noise_c0 + c, jnp.int32)])
            h = (nc * qj + tc) * m01
            plsc.store_scatter(
                out_v, [rows, jnp.full((L,), out_c0 + c, jnp.int32)], h)

    hyb(feat_v, FEAT, 0, 0, q[0])
    hyb(scal_v, 6, FEAT, FEAT, q[1])
    hyb(offs_v, OFFW, FEAT + 6, FEAT + 6, q[2])


def _sc_impl(anchor, feat, grid_scaling, grid_offsets, hyper_feat,
             noise_all, w_flat, idx):
    mesh = plsc.VectorSubcoreMesh(core_axis_name="c", subcore_axis_name="s")

    @functools.partial(
        pl.kernel,
        mesh=mesh,
        compiler_params=pltpu.CompilerParams(needs_layout_passes=False),
        out_type=jax.ShapeDtypeStruct((N, OUTW), jnp.float32),
        scratch_types=[
            pltpu.VMEM((WSLOTS,), jnp.int32),
            pltpu.VMEM((IW,), jnp.int32),
            pltpu.VMEM((CW,), jnp.int32),
            pltpu.VMEM((CW, 3), jnp.float32),
            pltpu.VMEM((CW, FEAT), jnp.float32),
            pltpu.VMEM((CW, 6), jnp.float32),
            pltpu.VMEM((CW, OFFW), jnp.float32),
            pltpu.VMEM((CW, FEAT), jnp.float32),
            pltpu.VMEM((CW, NOISE_W), jnp.float32),
            pltpu.VMEM((CW, OUTW), jnp.float32),
            pltpu.VMEM((512,), jnp.float32),
            pltpu.SemaphoreType.DMA,
        ],
    )
    def k(anchor_h, feat_h, scal_h, offs_h, hyper_h, noise_h, w_h, idx_h,
          out_h, winner_v, idxbuf_v, wstar_v, anchor_v, feat_v, scal_v,
          offs_v, hyper_v, noise_v, out_v, w_s, sem):
        wid = lax.axis_index("s") * NC + lax.axis_index("c")

        pltpu.sync_copy(w_h, w_s)

        def wini(i, carry):
            winner_v[pl.ds(i * L, L)] = jnp.full((L,), -1, jnp.int32)
            return carry
        lax.fori_loop(0, WSLOTS // L, wini, 0)

        def p1(w, carry):
            pltpu.sync_copy(idx_h.at[pl.ds(w * IW, IW)], idxbuf_v)

            def p1v(i, c2):
                v = idxbuf_v[pl.ds(i * L, L)]
                own = ((v >> 6) & 31) == wid
                slot = ((v >> 11) << 6) | (v & 63)
                b = w * IW + i * L + lax.iota(jnp.int32, L)
                plsc.store_scatter(winner_v, [slot], b, mask=own)
                return c2
            return lax.fori_loop(0, IW // L, p1v, carry)
        lax.fori_loop(0, NIW, p1, 0)

        def chunk(li, c, W):
            base = c * CW
            pltpu.sync_copy(anchor_h.at[pl.ds(base, W)], anchor_v.at[pl.ds(0, W)])
            pltpu.sync_copy(feat_h.at[pl.ds(base, W)], feat_v.at[pl.ds(0, W)])
            pltpu.sync_copy(scal_h.at[pl.ds(base, W)], scal_v.at[pl.ds(0, W)])
            pltpu.sync_copy(offs_h.at[pl.ds(base, W)], offs_v.at[pl.ds(0, W)])
            pltpu.sync_copy(hyper_h.at[pl.ds(base, W)], hyper_v.at[pl.ds(0, W)])

            def fw(i, carry):
                wv = winner_v[pl.ds(li * CW + i * L, L)]
                fallback = (base + i * L + lax.iota(jnp.int32, L)) & (2**17 - 1)
                wstar_v[pl.ds(i * L, L)] = jnp.where(wv >= 0, wv, fallback)
                return carry
            lax.fori_loop(0, W // L, fw, 0)
            for i in range(W // L, CW // L):
                wstar_v[pl.ds(i * L, L)] = (
                    base + i * L + lax.iota(jnp.int32, L)) & (2**17 - 1)

            pltpu.async_copy(noise_h.at[wstar_v], noise_v, sem).wait()

            def grp(g, carry):
                _group_body(g, li, winner_v, anchor_v, feat_v, scal_v,
                            offs_v, hyper_v, noise_v, out_v, w_s)
                return carry
            lax.fori_loop(0, W // L, grp, 0)
            pltpu.sync_copy(out_v.at[pl.ds(0, W)], out_h.at[pl.ds(base, W)])

        def p2(li, carry):
            c = li * NW + wid

            @pl.when(c < TAIL_C)
            def _():
                chunk(li, c, CW)

            @pl.when(c == TAIL_C)
            def _():
                chunk(li, c, TAIL_W)
            return carry
        lax.fori_loop(0, CPW, p2, 0)

    return k(anchor, feat, grid_scaling, grid_offsets, hyper_feat,
             noise_all, w_flat, idx)


def kernel(anchor, feat, grid_scaling, grid_offsets, hyper_feat,
           noise_feat, noise_scaling, noise_offsets, mlp_w, mlp_b, idx):
    noise_all = jnp.concatenate(
        [noise_feat, noise_scaling, noise_offsets,
         jnp.zeros((B, NOISE_W - OUTW), jnp.float32)], axis=1)
    w3 = mlp_w[:, 172:175]
    b3 = mlp_b[172:175][None, :]
    w_pad = jnp.concatenate([w3, b3], axis=0)
    w_pad = jnp.concatenate(
        [w_pad, jnp.zeros((110, 1), jnp.float32)], axis=1)
    w_flat = jnp.concatenate(
        [w_pad.reshape(440), jnp.zeros((72,), jnp.float32)])
    return _sc_impl(anchor, feat, grid_scaling, grid_offsets, hyper_feat,
                    noise_all, w_flat, idx.astype(jnp.int32))


# speedup vs baseline: 3.0426x; 1.3074x over previous
import functools

import jax
import jax.numpy as jnp
from jax import lax
from jax.experimental import pallas as pl
from jax.experimental.pallas import tpu as pltpu
from jax.experimental.pallas import tpu_sc as plsc

N = 500000
B = 250000
FEAT = 50
NOFF = 10
OFFW = 3 * NOFF
OUTW = FEAT + 6 + OFFW
IN_DIM = 9 + 2 * FEAT

NC = 2
NS = 16
NW = NC * NS
L = 16

CW = 48
NCHUNKS = (N + CW - 1) // CW
TAIL_C = NCHUNKS - 1
TAIL_W = N - TAIL_C * CW
TAIL_WID = TAIL_C % NW
CPW = (NCHUNKS + NW - 1) // NW
WSLOTS = CPW * CW

IW = 2000
NIW = B // IW

NOISE_W = 128


def _sc_impl(anchor, feat, grid_scaling, grid_offsets, hyper_feat,
             noise_all, w_flat, idx):
    mesh = plsc.VectorSubcoreMesh(core_axis_name="c", subcore_axis_name="s")

    @functools.partial(
        pl.kernel,
        mesh=mesh,
        compiler_params=pltpu.CompilerParams(needs_layout_passes=False),
        out_type=jax.ShapeDtypeStruct((N, OUTW), jnp.float32),
        scratch_types=[
            pltpu.VMEM((WSLOTS,), jnp.int32),
            pltpu.VMEM((2 * IW,), jnp.int32),
            pltpu.VMEM((2 * CW,), jnp.int32),
            pltpu.VMEM((2 * CW, 3), jnp.float32),
            pltpu.VMEM((2 * CW, FEAT), jnp.float32),
            pltpu.VMEM((2 * CW, 6), jnp.float32),
            pltpu.VMEM((2 * CW, OFFW), jnp.float32),
            pltpu.VMEM((2 * CW, FEAT), jnp.float32),
            pltpu.VMEM((2 * CW, NOISE_W), jnp.float32),
            pltpu.VMEM((CW, OUTW), jnp.float32),
            pltpu.VMEM((512,), jnp.float32),
            pltpu.SemaphoreType.DMA((2,)),
            pltpu.SemaphoreType.DMA,
        ],
    )
    def k(anchor_h, feat_h, scal_h, offs_h, hyper_h, noise_h, w_h, idx_h,
          out_h, winner_v, idxbuf_v, wstar_v, anchor_v, feat_v, scal_v,
          offs_v, hyper_v, noise_v, out_v, w_s, dsem, osem):
        wid = lax.axis_index("s") * NC + lax.axis_index("c")

        pltpu.sync_copy(w_h, w_s)

        def wini(i, carry):
            winner_v[pl.ds(i * L, L)] = jnp.full((L,), -1, jnp.int32)
            return carry
        lax.fori_loop(0, WSLOTS // L, wini, 0)

        def p1_issue(w, p):
            pltpu.async_copy(idx_h.at[pl.ds(w * IW, IW)],
                             idxbuf_v.at[pl.ds(p * IW, IW)], dsem.at[p])

        p1_issue(0, 0)

        def p1_step(w, pp):
            @pl.when(w + 1 < NIW)
            def _():
                p1_issue(w + 1, 1 - pp)
            pltpu.make_async_copy(idx_h.at[pl.ds(0, IW)],
                                  idxbuf_v.at[pl.ds(0, IW)],
                                  dsem.at[pp]).wait()

            def p1v(i, c2):
                v = idxbuf_v[pl.ds(pp * IW + i * L, L)]
                cch = ((v >> 4) * 21846) >> 16
                own = (cch & 31) == wid
                slot = (cch >> 5) * CW + (v - cch * CW)
                b = w * IW + i * L + lax.iota(jnp.int32, L)
                plsc.store_scatter(winner_v, [slot], b, mask=own)
                return c2
            lax.fori_loop(0, IW // L, p1v, 0)

        def p1pair(hw, carry):
            p1_step(2 * hw, 0)
            p1_step(2 * hw + 1, 1)
            return carry
        lax.fori_loop(0, NIW // 2, p1pair, 0)
        p1_step(NIW - 1, 0)

        def fill_wstar(li, base, p, nrows):
            def fw(i, carry):
                wv = winner_v[pl.ds(li * CW + i * L, L)]
                fallback = (base + i * L + lax.iota(jnp.int32, L)) & (2**17 - 1)
                wstar_v[pl.ds(p * CW + i * L, L)] = jnp.where(wv >= 0, wv,
                                                              fallback)
                return carry
            lax.fori_loop(0, nrows // L, fw, 0)
            for i in range(nrows // L, CW // L):
                wstar_v[pl.ds(p * CW + i * L, L)] = (
                    base + i * L + lax.iota(jnp.int32, L)) & (2**17 - 1)

        def issue(li, p, W):
            c = li * NW + wid
            base = c * CW
            off = p * CW
            pltpu.async_copy(anchor_h.at[pl.ds(base, W)],
                             anchor_v.at[pl.ds(off, W)], dsem.at[p])
            pltpu.async_copy(feat_h.at[pl.ds(base, W)],
                             feat_v.at[pl.ds(off, W)], dsem.at[p])
            pltpu.async_copy(scal_h.at[pl.ds(base, W)],
                             scal_v.at[pl.ds(off, W)], dsem.at[p])
            pltpu.async_copy(offs_h.at[pl.ds(base, W)],
                             offs_v.at[pl.ds(off, W)], dsem.at[p])
            pltpu.async_copy(hyper_h.at[pl.ds(base, W)],
                             hyper_v.at[pl.ds(off, W)], dsem.at[p])
            fill_wstar(li, base, p, W)
            pltpu.async_copy(noise_h.at[wstar_v.at[pl.ds(off, CW)]],
                             noise_v.at[pl.ds(off, CW)], dsem.at[p])

        def drain(p, W):
            for src, dst in ((anchor_h, anchor_v), (feat_h, feat_v),
                             (scal_h, scal_v), (offs_h, offs_v),
                             (hyper_h, hyper_v)):
                pltpu.make_async_copy(src.at[pl.ds(0, W)],
                                      dst.at[pl.ds(0, W)], dsem.at[p]).wait()
            off = p * CW
            pltpu.make_async_copy(noise_h.at[wstar_v.at[pl.ds(off, CW)]],
                                  noise_v.at[pl.ds(off, CW)],
                                  dsem.at[p]).wait()

        def wait_out():
            pltpu.make_async_copy(out_v, out_h.at[pl.ds(0, CW)], osem).wait()

        def compute(li, p, W):
            c = li * NW + wid
            off = p * CW

            def grp(g, carry):
                rows_t = off + g * L + lax.iota(jnp.int32, L)
                rows_o = g * L + lax.iota(jnp.int32, L)
                wb = w_s[pl.ds(4 * IN_DIM, L)]
                acc = [jnp.zeros((L,), jnp.float32) + wb[j] for j in range(3)]

                def dot_tbl(tbl, ncols, wrow0):
                    for kk in range(ncols):
                        col = plsc.load_gather(
                            tbl, [rows_t, jnp.full((L,), kk, jnp.int32)])
                        wk = w_s[pl.ds(4 * (wrow0 + kk), L)]
                        for j in range(3):
                            acc[j] = col * wk[j] + acc[j]

                dot_tbl(anchor_v, 3, 0)
                dot_tbl(feat_v, FEAT, 3)
                dot_tbl(scal_v, 6, 3 + FEAT)
                dot_tbl(hyper_v, FEAT, 9 + FEAT)

                wv = winner_v[pl.ds(li * CW + g * L, L)]
                m01 = jnp.where(wv >= 0, 1.0, 0.0).astype(jnp.float32)
                q = []
                for j, scale in ((0, 1.0), (1, 0.001), (2, 0.2)):
                    t = 1.0 - 2.0 / (jnp.exp(acc[j] * 2.0) + 1.0)
                    q.append(jnp.maximum(scale * (1.0 + t), 1e-9))

                def hyb(tbl, ncols, noise_c0, out_c0, qj):
                    for cc in range(ncols):
                        tc = plsc.load_gather(
                            tbl, [rows_t, jnp.full((L,), cc, jnp.int32)])
                        nc = plsc.load_gather(
                            noise_v,
                            [rows_t, jnp.full((L,), noise_c0 + cc, jnp.int32)])
                        h = (nc * qj + tc) * m01
                        plsc.store_scatter(
                            out_v,
                            [rows_o, jnp.full((L,), out_c0 + cc, jnp.int32)],
                            h)

                hyb(feat_v, FEAT, 0, 0, q[0])
                hyb(scal_v, 6, FEAT, FEAT, q[1])
                hyb(offs_v, OFFW, FEAT + 6, FEAT + 6, q[2])
                return carry
            lax.fori_loop(0, W // L, grp, 0)
            pltpu.async_copy(out_v.at[pl.ds(0, W)],
                             out_h.at[pl.ds(c * CW, W)], osem)

        def issue_next(li, p):
            cn = (li + 1) * NW + wid

            @pl.when(cn < TAIL_C)
            def _():
                issue(li + 1, 1 - p, CW)

            @pl.when(cn == TAIL_C)
            def _():
                issue(li + 1, 1 - p, TAIL_W)

        issue(0, 0, CW)

        def p2_step(li, pp):
            c = li * NW + wid

            @pl.when(c < TAIL_C)
            def _():
                issue_next(li, pp)
                drain(pp, CW)

                @pl.when(li > 0)
                def _():
                    wait_out()
                compute(li, pp, CW)

            @pl.when(c == TAIL_C)
            def _():
                issue_next(li, pp)
                drain(pp, TAIL_W)

                @pl.when(li > 0)
                def _():
                    wait_out()
                compute(li, pp, TAIL_W)

        def p2pair(h, carry):
            p2_step(2 * h, 0)
            p2_step(2 * h + 1, 1)
            return carry
        lax.fori_loop(0, CPW // 2, p2pair, 0)

        @pl.when(wid != TAIL_WID)
        def _():
            pltpu.make_async_copy(out_v, out_h.at[pl.ds(0, CW)], osem).wait()

        @pl.when(wid == TAIL_WID)
        def _():
            pltpu.make_async_copy(out_v.at[pl.ds(0, TAIL_W)],
                                  out_h.at[pl.ds(0, TAIL_W)], osem).wait()

    return k(anchor, feat, grid_scaling, grid_offsets, hyper_feat,
             noise_all, w_flat, idx)


def kernel(anchor, feat, grid_scaling, grid_offsets, hyper_feat,
           noise_feat, noise_scaling, noise_offsets, mlp_w, mlp_b, idx):
    noise_all = jnp.concatenate(
        [noise_feat, noise_scaling, noise_offsets,
         jnp.zeros((B, NOISE_W - OUTW), jnp.float32)], axis=1)
    w3 = mlp_w[:, 172:175]
    b3 = mlp_b[172:175][None, :]
    w_pad = jnp.concatenate([w3, b3], axis=0)
    w_pad = jnp.concatenate(
        [w_pad, jnp.zeros((110, 1), jnp.float32)], axis=1)
    w_flat = jnp.concatenate(
        [w_pad.reshape(440), jnp.zeros((72,), jnp.float32)])
    return _sc_impl(anchor, feat, grid_scaling, grid_offsets, hyper_feat,
                    noise_all, w_flat, idx.astype(jnp.int32))
